# Initial kernel scaffold; baseline (speedup 1.0000x reference)
import jax
import jax.numpy as jnp
from jax.experimental import pallas as pl


def kernel(inputs, embedding):
    raise NotImplementedError("write your pallas kernel here")


# speedup vs baseline: 1.0936x; 1.0936x over previous
import functools

import jax
import jax.numpy as jnp
from jax import lax
from jax.experimental import pallas as pl
from jax.experimental.pallas import tpu as pltpu
from jax.experimental.pallas import tpu_sc as plsc

NUM_CORES = 2
NUM_SUBCORES = 16
NW = NUM_CORES * NUM_SUBCORES

B_ROWS = 16384
SEQ = 50
FEATURES = 32
B = B_ROWS * SEQ
IDX_MINOR = 128
CHUNK_VECS = 8
CHUNK = CHUNK_VECS * IDX_MINOR
ROWS_PER_W = B // NW
CHUNKS_PER_W = ROWS_PER_W // CHUNK


def _body(idx_hbm, table_hbm, out_hbm, idx_v, rows_v, in_sem, gat_sem):
    wid = lax.axis_index("s") * NUM_CORES + lax.axis_index("c")
    idx_row0 = wid * (ROWS_PER_W // IDX_MINOR)
    out_base = wid * ROWS_PER_W

    def chunk(j, carry):
        pltpu.sync_copy(idx_hbm.at[pl.ds(idx_row0 + j * CHUNK_VECS, CHUNK_VECS)],
                        idx_v)
        waits = []
        for v in range(CHUNK_VECS):
            waits.append(
                pltpu.async_copy(table_hbm.at[idx_v.at[v]],
                                 rows_v.at[pl.ds(v * IDX_MINOR, IDX_MINOR)],
                                 gat_sem))
        for w in waits:
            w.wait()
        pltpu.sync_copy(rows_v, out_hbm.at[pl.ds(out_base + j * CHUNK, CHUNK)])
        return carry

    lax.fori_loop(0, CHUNKS_PER_W, chunk, 0)


@functools.partial(jax.jit, static_argnames=())
def _lookup(idx2d, embedding):
    mesh = plsc.VectorSubcoreMesh(core_axis_name="c", subcore_axis_name="s")
    run = pl.kernel(
        _body,
        out_type=jax.ShapeDtypeStruct((B, FEATURES), jnp.float32),
        mesh=mesh,
        scratch_types=[
            pltpu.VMEM((CHUNK_VECS, IDX_MINOR), jnp.int32),
            pltpu.VMEM((CHUNK, FEATURES), jnp.float32),
            pltpu.SemaphoreType.DMA,
            pltpu.SemaphoreType.DMA,
        ],
        compiler_params=pltpu.CompilerParams(use_tc_tiling_on_sc=False),
    )
    return run(idx2d, embedding)


def kernel(inputs, embedding):
    idx2d = inputs.astype(jnp.int32).reshape(B // IDX_MINOR, IDX_MINOR)
    out = _lookup(idx2d, embedding)
    return out.reshape(B_ROWS, SEQ, FEATURES)


# speedup vs baseline: 1.1093x; 1.0144x over previous
import functools

import jax
import jax.numpy as jnp
from jax import lax
from jax.experimental import pallas as pl
from jax.experimental.pallas import tpu as pltpu
from jax.experimental.pallas import tpu_sc as plsc

NUM_CORES = 2
NUM_SUBCORES = 16
NW = NUM_CORES * NUM_SUBCORES

B_ROWS = 16384
SEQ = 50
FEATURES = 32
B = B_ROWS * SEQ
IDX_MINOR = 128
CHUNK_VECS = 10
CHUNK = CHUNK_VECS * IDX_MINOR
ROWS_PER_W = B // NW
NCHUNK = ROWS_PER_W // CHUNK
NBUF = 2


def _body(idx_hbm, table_hbm, out_hbm, idx_v, rows_v, idx_sem, gat_sem,
          out_sem):
    wid = lax.axis_index("s") * NUM_CORES + lax.axis_index("c")
    idx_row0 = wid * (ROWS_PER_W // IDX_MINOR)
    out_base = wid * ROWS_PER_W

    def start_idx(j, b):
        pltpu.make_async_copy(
            idx_hbm.at[pl.ds(idx_row0 + j * CHUNK_VECS, CHUNK_VECS)],
            idx_v.at[b], idx_sem).start()

    def wait_idx(b):
        pltpu.make_async_copy(
            idx_hbm.at[pl.ds(idx_row0, CHUNK_VECS)], idx_v.at[b],
            idx_sem).wait()

    def gather(b):
        waits = []
        for v in range(CHUNK_VECS):
            waits.append(
                pltpu.async_copy(table_hbm.at[idx_v.at[b, v]],
                                 rows_v.at[b, pl.ds(v * IDX_MINOR, IDX_MINOR)],
                                 gat_sem))
        for w in waits:
            w.wait()

    def start_out(j, b):
        pltpu.make_async_copy(
            rows_v.at[b], out_hbm.at[pl.ds(out_base + j * CHUNK, CHUNK)],
            out_sem).start()

    def wait_out(b):
        pltpu.make_async_copy(
            rows_v.at[b], out_hbm.at[pl.ds(out_base, CHUNK)], out_sem).wait()

    for b in range(NBUF):
        start_idx(b, b)
    for b in range(NBUF):
        wait_idx(b)
        gather(b)
        start_idx(b + NBUF, b)
        start_out(b, b)

    def group(g, carry):
        for b in range(NBUF):
            j = NBUF * g + b
            wait_out(b)
            wait_idx(b)
            gather(b)
            @pl.when(j + NBUF < NCHUNK)
            def _():
                start_idx(j + NBUF, b)
            start_out(j, b)
        return carry

    lax.fori_loop(1, NCHUNK // NBUF, group, 0)

    for b in range(NBUF):
        wait_out(b)


@jax.jit
def _lookup(idx2d, embedding):
    mesh = plsc.VectorSubcoreMesh(core_axis_name="c", subcore_axis_name="s")
    run = pl.kernel(
        _body,
        out_type=jax.ShapeDtypeStruct((B, FEATURES), jnp.float32),
        mesh=mesh,
        scratch_types=[
            pltpu.VMEM((NBUF, CHUNK_VECS, IDX_MINOR), jnp.int32),
            pltpu.VMEM((NBUF, CHUNK, FEATURES), jnp.float32),
            pltpu.SemaphoreType.DMA,
            pltpu.SemaphoreType.DMA,
            pltpu.SemaphoreType.DMA,
        ],
        compiler_params=pltpu.CompilerParams(use_tc_tiling_on_sc=False),
    )
    return run(idx2d, embedding)


def kernel(inputs, embedding):
    idx2d = inputs.astype(jnp.int32).reshape(B // IDX_MINOR, IDX_MINOR)
    out = _lookup(idx2d, embedding)
    return out.reshape(B_ROWS, SEQ, FEATURES)


# speedup vs baseline: 1.1122x; 1.0027x over previous
import functools

import jax
import jax.numpy as jnp
from jax import lax
from jax.experimental import pallas as pl
from jax.experimental.pallas import tpu as pltpu
from jax.experimental.pallas import tpu_sc as plsc

NUM_CORES = 2
NUM_SUBCORES = 16
NW = NUM_CORES * NUM_SUBCORES

B_ROWS = 16384
SEQ = 50
FEATURES = 32
B = B_ROWS * SEQ
IDX_MINOR = 128
CHUNK_VECS = 10
CHUNK = CHUNK_VECS * IDX_MINOR
ROWS_PER_W = B // NW
NCHUNK = ROWS_PER_W // CHUNK
NBUF = 2


def _body(idx_hbm, table_hbm, out_hbm, idx_v, rows_v, idx_sem, gat_sem,
          out_sem):
    wid = lax.axis_index("s") * NUM_CORES + lax.axis_index("c")
    idx_row0 = wid * (ROWS_PER_W // IDX_MINOR)
    out_base = wid * ROWS_PER_W

    def start_idx(j, b):
        pltpu.make_async_copy(
            idx_hbm.at[pl.ds(idx_row0 + j * CHUNK_VECS, CHUNK_VECS)],
            idx_v.at[b], idx_sem).start()

    def wait_idx(b):
        pltpu.make_async_copy(
            idx_hbm.at[pl.ds(idx_row0, CHUNK_VECS)], idx_v.at[b],
            idx_sem).wait()

    def start_out(j, b):
        pltpu.make_async_copy(
            rows_v.at[b], out_hbm.at[pl.ds(out_base + j * CHUNK, CHUNK)],
            out_sem).start()

    def wait_out(b):
        pltpu.make_async_copy(
            rows_v.at[b], out_hbm.at[pl.ds(out_base, CHUNK)], out_sem).wait()

    def gather_fire(j, b):
        for v in range(CHUNK_VECS):
            pltpu.async_copy(table_hbm.at[idx_v.at[b, v]],
                             rows_v.at[b, pl.ds(v * IDX_MINOR, IDX_MINOR)],
                             gat_sem)

    def gather_wait(b):
        for v in range(CHUNK_VECS):
            pltpu.make_async_copy(
                table_hbm.at[idx_v.at[b, v]],
                rows_v.at[b, pl.ds(v * IDX_MINOR, IDX_MINOR)],
                gat_sem).wait()

    for b in range(NBUF):
        start_idx(b, b)
    wait_idx(0)
    gather_fire(0, 0)

    def group(g, carry):
        for b in range(NBUF):
            j = NBUF * g + b
            nb = 1 - b

            @pl.when(j >= 1)
            def _(nb=nb):
                wait_out(nb)

            @pl.when(j + 1 < NCHUNK)
            def _(j=j, nb=nb):
                wait_idx(nb)
                gather_fire(j + 1, nb)

            gather_wait(b)

            @pl.when(j + 2 < NCHUNK)
            def _(j=j, b=b):
                start_idx(j + 2, b)

            start_out(j, b)
        return carry

    lax.fori_loop(0, NCHUNK // NBUF, group, 0)

    wait_out((NCHUNK - 1) % NBUF)


@jax.jit
def _lookup(idx2d, embedding):
    mesh = plsc.VectorSubcoreMesh(core_axis_name="c", subcore_axis_name="s")
    run = pl.kernel(
        _body,
        out_type=jax.ShapeDtypeStruct((B, FEATURES), jnp.float32),
        mesh=mesh,
        scratch_types=[
            pltpu.VMEM((NBUF, CHUNK_VECS, IDX_MINOR), jnp.int32),
            pltpu.VMEM((NBUF, CHUNK, FEATURES), jnp.float32),
            pltpu.SemaphoreType.DMA,
            pltpu.SemaphoreType.DMA,
            pltpu.SemaphoreType.DMA,
        ],
        compiler_params=pltpu.CompilerParams(use_tc_tiling_on_sc=False),
    )
    return run(idx2d, embedding)


def kernel(inputs, embedding):
    idx2d = inputs.astype(jnp.int32).reshape(B // IDX_MINOR, IDX_MINOR)
    out = _lookup(idx2d, embedding)
    return out.reshape(B_ROWS, SEQ, FEATURES)


# speedup vs baseline: 1.8029x; 1.6210x over previous
import jax
import jax.numpy as jnp
from jax import lax
from jax.experimental import pallas as pl
from jax.experimental.pallas import tpu as pltpu
from jax.experimental.pallas import tpu_sc as plsc

NUM_CORES = 2
NUM_SUBCORES = 16
NW = NUM_CORES * NUM_SUBCORES

B_ROWS = 16384
SEQ = 50
FEATURES = 32
ROWS_PER_W = B_ROWS // NW
CH_ROWS = 16
NCHUNK = ROWS_PER_W // CH_ROWS
NBUF = 2


def _body(idx_hbm, table_hbm, out_hbm, idx_v, rows_v, idx_sem, gat_sem,
          out_sem):
    wid = lax.axis_index("s") * NUM_CORES + lax.axis_index("c")
    row0 = wid * ROWS_PER_W

    def start_idx(j, b):
        pltpu.make_async_copy(
            idx_hbm.at[pl.ds(row0 + j * CH_ROWS, CH_ROWS)],
            idx_v.at[b], idx_sem).start()

    def wait_idx(b):
        pltpu.make_async_copy(
            idx_hbm.at[pl.ds(row0, CH_ROWS)], idx_v.at[b], idx_sem).wait()

    def gather_fire(b):
        for r in range(CH_ROWS):
            pltpu.async_copy(table_hbm.at[idx_v.at[b, r]],
                             rows_v.at[b, r], gat_sem)

    def gather_wait(b):
        for r in range(CH_ROWS):
            pltpu.make_async_copy(table_hbm.at[idx_v.at[b, r]],
                                  rows_v.at[b, r], gat_sem).wait()

    def start_out(j, b):
        pltpu.make_async_copy(
            rows_v.at[b],
            out_hbm.at[pl.ds(row0 + j * CH_ROWS, CH_ROWS)],
            out_sem).start()

    def wait_out(b):
        pltpu.make_async_copy(
            rows_v.at[b], out_hbm.at[pl.ds(row0, CH_ROWS)], out_sem).wait()

    for b in range(NBUF):
        start_idx(b, b)
    wait_idx(0)
    gather_fire(0)

    def group(g, carry):
        for b in range(NBUF):
            j = NBUF * g + b
            nb = 1 - b

            @pl.when(j >= 1)
            def _(nb=nb):
                wait_out(nb)

            @pl.when(j + 1 < NCHUNK)
            def _(nb=nb):
                wait_idx(nb)
                gather_fire(nb)

            gather_wait(b)

            @pl.when(j + 2 < NCHUNK)
            def _(j=j, b=b):
                start_idx(j + 2, b)

            start_out(j, b)
        return carry

    lax.fori_loop(0, NCHUNK // NBUF, group, 0)

    wait_out((NCHUNK - 1) % NBUF)


@jax.jit
def _lookup(idx, embedding):
    mesh = plsc.VectorSubcoreMesh(core_axis_name="c", subcore_axis_name="s")
    run = pl.kernel(
        _body,
        out_type=jax.ShapeDtypeStruct((B_ROWS, SEQ, FEATURES), jnp.float32),
        mesh=mesh,
        scratch_types=[
            pltpu.VMEM((NBUF, CH_ROWS, SEQ), jnp.int32),
            pltpu.VMEM((NBUF, CH_ROWS, SEQ, FEATURES), jnp.float32),
            pltpu.SemaphoreType.DMA,
            pltpu.SemaphoreType.DMA,
            pltpu.SemaphoreType.DMA,
        ],
        compiler_params=pltpu.CompilerParams(use_tc_tiling_on_sc=False),
    )
    return run(idx, embedding)


def kernel(inputs, embedding):
    return _lookup(inputs.astype(jnp.int32), embedding)
